# baseline (speedup 1.0000x reference)
import functools

import jax
import jax.numpy as jnp
from jax import lax
from jax.experimental import pallas as pl
from jax.experimental.pallas import tpu as pltpu
from jax.experimental.pallas import tpu_sc as plsc

N = 10000
E = 320000
D = 128
T = 4
STEPS = 6
G = 128

NTILES = 32
NPAD = 10240
ROWS_PER_TILE = NPAD // 16
E_PAD = 327680
CHUNK = 64
NCHTOT = E_PAD // CHUNK
NCH_H = 320
NCH_L = 0
HEAVY_CORE = 0
SUPER = 64

_HIGH = jax.lax.Precision.HIGHEST


def _sc_body(ht_hbm, gix_hbm, dix_hbm, zz_hbm, m2_hbm, gv, dv, rv0, rv1,
             msh, sem0, sem1):
    c = lax.axis_index("c")
    s = lax.axis_index("s")
    pltpu.sync_copy(zz_hbm, rv0)

    def zstep(i, carry):
        pltpu.sync_copy(rv0, msh.at[pl.ds(s * ROWS_PER_TILE + i * CHUNK, CHUNK)])
        return carry

    lax.fori_loop(0, ROWS_PER_TILE // CHUNK, zstep, 0)
    plsc.subcore_barrier()

    heavy = c == HEAVY_CORE
    nsup = jnp.where(heavy, NCH_H // SUPER, NCH_L // SUPER)
    base = jnp.where(heavy, s * NCH_H, 16 * NCH_H + s * NCH_L)

    def super_step(u, carry):
        row0 = base + u * SUPER
        pltpu.sync_copy(gix_hbm.at[pl.ds(row0, SUPER)], gv)
        pltpu.sync_copy(dix_hbm.at[pl.ds(row0, SUPER)], dv)
        pltpu.async_copy(ht_hbm.at[gv.at[0]], rv0, sem0)

        def step(k, carry2):
            j0 = 2 * k
            j1 = 2 * k + 1
            pltpu.make_async_copy(ht_hbm.at[gv.at[j0]], rv0, sem0).wait()
            pltpu.async_copy(ht_hbm.at[gv.at[j1]], rv1, sem1)
            pltpu.sync_copy(rv0, msh.at[dv.at[j0]], add=True)
            pltpu.make_async_copy(ht_hbm.at[gv.at[j1]], rv1, sem1).wait()

            @pl.when(k < SUPER // 2 - 1)
            def _():
                pltpu.async_copy(ht_hbm.at[gv.at[j0 + 2]], rv0, sem0)

            pltpu.sync_copy(rv1, msh.at[dv.at[j1]], add=True)
            return carry2

        lax.fori_loop(0, SUPER // 2, step, 0)
        return carry

    lax.fori_loop(0, nsup, super_step, 0)
    plsc.subcore_barrier()
    pltpu.sync_copy(
        msh.at[pl.ds(s * ROWS_PER_TILE, ROWS_PER_TILE)],
        m2_hbm.at[c, pl.ds(s * ROWS_PER_TILE, ROWS_PER_TILE)],
    )


@functools.cache
def _get_sc_scatter():
    mesh = plsc.VectorSubcoreMesh(core_axis_name="c", subcore_axis_name="s")
    return pl.kernel(
        _sc_body,
        out_type=jax.ShapeDtypeStruct((2, NPAD, D), jnp.float32),
        mesh=mesh,
        scratch_types=[
            pltpu.VMEM((SUPER, CHUNK), jnp.int32),
            pltpu.VMEM((SUPER, CHUNK), jnp.int32),
            pltpu.VMEM((CHUNK, D), jnp.float32),
            pltpu.VMEM((CHUNK, D), jnp.float32),
            pltpu.VMEM_SHARED((NPAD, D), jnp.float32),
            pltpu.SemaphoreType.DMA,
            pltpu.SemaphoreType.DMA,
        ],
    )


BN = 1024


def _tc_body(with_gru, with_ht, *refs):
    if with_gru:
        m2_ref, h_ref, wih_ref, whh_ref, bih_ref, bhh_ref = refs[:6]
        refs = refs[6:]
    else:
        h_ref = refs[0]
        refs = refs[1:]
    if with_ht:
        we_ref = refs[0]
        refs = refs[1:]
    h = h_ref[...]
    if with_gru:
        m = m2_ref[0] + m2_ref[1]
        gx = jnp.dot(m, wih_ref[...],
                     preferred_element_type=jnp.float32) + bih_ref[...]
        gh = jnp.dot(h, whh_ref[...],
                     preferred_element_type=jnp.float32) + bhh_ref[...]
        r = jax.nn.sigmoid(gx[:, :D] + gh[:, :D])
        z = jax.nn.sigmoid(gx[:, D:2 * D] + gh[:, D:2 * D])
        n = jnp.tanh(gx[:, 2 * D:] + r * gh[:, 2 * D:])
        h = (1.0 - z) * n + z * h
        hout_ref = refs[0]
        refs = refs[1:]
        hout_ref[...] = h
    if with_ht:
        htout_ref = refs[0]
        for e in range(T):
            htout_ref[e] = jnp.dot(h, we_ref[e],
                                   preferred_element_type=jnp.float32)


def _make_tc_step(with_gru, with_ht):
    grid = (NPAD // BN,)
    in_specs = []
    if with_gru:
        in_specs += [
            pl.BlockSpec((2, BN, D), lambda i: (0, i, 0)),
            pl.BlockSpec((BN, D), lambda i: (i, 0)),
            pl.BlockSpec((D, 3 * D), lambda i: (0, 0)),
            pl.BlockSpec((D, 3 * D), lambda i: (0, 0)),
            pl.BlockSpec((1, 3 * D), lambda i: (0, 0)),
            pl.BlockSpec((1, 3 * D), lambda i: (0, 0)),
        ]
    else:
        in_specs += [pl.BlockSpec((BN, D), lambda i: (i, 0))]
    if with_ht:
        in_specs += [pl.BlockSpec((T, D, D), lambda i: (0, 0, 0))]
    out_specs = []
    out_shapes = []
    if with_gru:
        out_specs.append(pl.BlockSpec((BN, D), lambda i: (i, 0)))
        out_shapes.append(jax.ShapeDtypeStruct((NPAD, D), jnp.float32))
    if with_ht:
        out_specs.append(pl.BlockSpec((T, BN, D), lambda i: (0, i, 0)))
        out_shapes.append(jax.ShapeDtypeStruct((T, NPAD, D), jnp.float32))
    if len(out_shapes) == 1:
        out_specs, out_shapes = out_specs[0], out_shapes[0]
    return pl.pallas_call(
        functools.partial(_tc_body, with_gru, with_ht),
        grid=grid,
        in_specs=in_specs,
        out_specs=out_specs,
        out_shape=out_shapes,
    )


_tc_ht0 = _make_tc_step(False, True)
_tc_gru_ht = _make_tc_step(True, True)
_tc_gru = _make_tc_step(True, False)

BNR = 640


def _readout_body(gid_ref, h_ref, out_ref):
    i = pl.program_id(0)
    ids = gid_ref[0]
    iot = lax.broadcasted_iota(jnp.int32, (G, BNR), 0)
    oh = (iot == ids).astype(jnp.float32)
    contrib = lax.dot_general(oh, h_ref[...], (((1,), (0,)), ((), ())),
                              precision=_HIGH,
                              preferred_element_type=jnp.float32)

    @pl.when(i == 0)
    def _():
        out_ref[...] = contrib

    @pl.when(i != 0)
    def _():
        out_ref[...] += contrib


_tc_readout = pl.pallas_call(
    _readout_body,
    grid=(NPAD // BNR,),
    in_specs=[
        pl.BlockSpec((1, 1, BNR), lambda i: (i, 0, 0)),
        pl.BlockSpec((BNR, D), lambda i: (i, 0)),
    ],
    out_specs=pl.BlockSpec((G, D), lambda i: (0, 0)),
    out_shape=jax.ShapeDtypeStruct((G, D), jnp.float32),
)


def kernel(features, edge_index, edge_types, graph_ids, W_edge, W_ih, W_hh, b_ih, b_hh):
    src = edge_index[0]
    dst = edge_index[1]
    pad_e = E_PAD - E
    gidx = edge_types * NPAD + src
    gidx = jnp.concatenate([gidx, jnp.full((pad_e,), N, jnp.int32)])
    didx = jnp.concatenate([dst, jnp.full((pad_e,), N, jnp.int32)])
    gidx3 = gidx.reshape(NCHTOT, CHUNK)
    didx3 = didx.reshape(NCHTOT, CHUNK)

    h = jnp.zeros((NPAD, D), jnp.float32).at[:N].set(features)
    zblk = jnp.zeros((CHUNK, D), jnp.float32)
    b_ih2 = b_ih.reshape(1, 3 * D)
    b_hh2 = b_hh.reshape(1, 3 * D)
    gid3 = (jnp.full((NPAD,), -1, jnp.int32).at[:N].set(graph_ids)
            .reshape(NPAD // BNR, 1, BNR))

    sc_scatter = _get_sc_scatter()
    ht = _tc_ht0(h, W_edge)
    for step in range(STEPS):
        m2 = sc_scatter(ht.reshape(T * NPAD, D), gidx3, didx3, zblk)
        if step < STEPS - 1:
            h, ht = _tc_gru_ht(m2, h, W_ih, W_hh, b_ih2, b_hh2, W_edge)
        else:
            h = _tc_gru(m2, h, W_ih, W_hh, b_ih2, b_hh2)
    return _tc_readout(gid3, h)


# speedup vs baseline: 1.5878x; 1.5878x over previous
import functools

import jax
import jax.numpy as jnp
from jax import lax
from jax.experimental import pallas as pl
from jax.experimental.pallas import tpu as pltpu
from jax.experimental.pallas import tpu_sc as plsc

N = 10000
E = 320000
D = 128
T = 4
STEPS = 6
G = 128

NTILES = 32
NPAD = 10240
ROWS_PER_TILE = NPAD // 16
E_PAD = 327680
CHUNK = 64
NCHTOT = E_PAD // CHUNK
NCH_H = 280
NCH_L = 40
HEAVY_CORE = 0
SUPER = 40

_HIGH = jax.lax.Precision.HIGHEST


def _sc_body(ht_hbm, gix_hbm, dix_hbm, zz_hbm, m2_hbm, gv, dv, rv0, rv1,
             msh, sem0, sem1):
    c = lax.axis_index("c")
    s = lax.axis_index("s")
    pltpu.sync_copy(zz_hbm, rv0)

    def zstep(i, carry):
        pltpu.sync_copy(rv0, msh.at[pl.ds(s * ROWS_PER_TILE + i * CHUNK, CHUNK)])
        return carry

    lax.fori_loop(0, ROWS_PER_TILE // CHUNK, zstep, 0)
    plsc.subcore_barrier()

    heavy = c == HEAVY_CORE
    nsup = jnp.where(heavy, NCH_H // SUPER, NCH_L // SUPER)
    base = jnp.where(heavy, s * NCH_H, 16 * NCH_H + s * NCH_L)

    def super_step(u, carry):
        row0 = base + u * SUPER
        pltpu.sync_copy(gix_hbm.at[pl.ds(row0, SUPER)], gv)
        pltpu.sync_copy(dix_hbm.at[pl.ds(row0, SUPER)], dv)
        pltpu.async_copy(ht_hbm.at[gv.at[0]], rv0, sem0)

        def step(k, carry2):
            j0 = 2 * k
            j1 = 2 * k + 1
            pltpu.make_async_copy(ht_hbm.at[gv.at[j0]], rv0, sem0).wait()
            pltpu.async_copy(ht_hbm.at[gv.at[j1]], rv1, sem1)
            pltpu.sync_copy(rv0, msh.at[dv.at[j0]], add=True)
            pltpu.make_async_copy(ht_hbm.at[gv.at[j1]], rv1, sem1).wait()

            @pl.when(k < SUPER // 2 - 1)
            def _():
                pltpu.async_copy(ht_hbm.at[gv.at[j0 + 2]], rv0, sem0)

            pltpu.sync_copy(rv1, msh.at[dv.at[j1]], add=True)
            return carry2

        lax.fori_loop(0, SUPER // 2, step, 0)
        return carry

    lax.fori_loop(0, nsup, super_step, 0)
    plsc.subcore_barrier()
    pltpu.sync_copy(
        msh.at[pl.ds(s * ROWS_PER_TILE, ROWS_PER_TILE)],
        m2_hbm.at[c, pl.ds(s * ROWS_PER_TILE, ROWS_PER_TILE)],
    )


@functools.cache
def _get_sc_scatter():
    mesh = plsc.VectorSubcoreMesh(core_axis_name="c", subcore_axis_name="s")
    return pl.kernel(
        _sc_body,
        out_type=jax.ShapeDtypeStruct((2, NPAD, D), jnp.float32),
        mesh=mesh,
        scratch_types=[
            pltpu.VMEM((SUPER, CHUNK), jnp.int32),
            pltpu.VMEM((SUPER, CHUNK), jnp.int32),
            pltpu.VMEM((CHUNK, D), jnp.float32),
            pltpu.VMEM((CHUNK, D), jnp.float32),
            pltpu.VMEM_SHARED((NPAD, D), jnp.float32),
            pltpu.SemaphoreType.DMA,
            pltpu.SemaphoreType.DMA,
        ],
    )


BN = 1024


def _tc_body(with_gru, with_ht, *refs):
    if with_gru:
        m2_ref, h_ref, wih_ref, whh_ref, bih_ref, bhh_ref = refs[:6]
        refs = refs[6:]
    else:
        h_ref = refs[0]
        refs = refs[1:]
    if with_ht:
        we_ref = refs[0]
        refs = refs[1:]
    h = h_ref[...]
    if with_gru:
        m = m2_ref[0] + m2_ref[1]
        gx = jnp.dot(m, wih_ref[...],
                     preferred_element_type=jnp.float32) + bih_ref[...]
        gh = jnp.dot(h, whh_ref[...],
                     preferred_element_type=jnp.float32) + bhh_ref[...]
        r = jax.nn.sigmoid(gx[:, :D] + gh[:, :D])
        z = jax.nn.sigmoid(gx[:, D:2 * D] + gh[:, D:2 * D])
        n = jnp.tanh(gx[:, 2 * D:] + r * gh[:, 2 * D:])
        h = (1.0 - z) * n + z * h
        hout_ref = refs[0]
        refs = refs[1:]
        hout_ref[...] = h
    if with_ht:
        htout_ref = refs[0]
        for e in range(T):
            htout_ref[e] = jnp.dot(h, we_ref[e],
                                   preferred_element_type=jnp.float32)


def _make_tc_step(with_gru, with_ht):
    grid = (NPAD // BN,)
    in_specs = []
    if with_gru:
        in_specs += [
            pl.BlockSpec((2, BN, D), lambda i: (0, i, 0)),
            pl.BlockSpec((BN, D), lambda i: (i, 0)),
            pl.BlockSpec((D, 3 * D), lambda i: (0, 0)),
            pl.BlockSpec((D, 3 * D), lambda i: (0, 0)),
            pl.BlockSpec((1, 3 * D), lambda i: (0, 0)),
            pl.BlockSpec((1, 3 * D), lambda i: (0, 0)),
        ]
    else:
        in_specs += [pl.BlockSpec((BN, D), lambda i: (i, 0))]
    if with_ht:
        in_specs += [pl.BlockSpec((T, D, D), lambda i: (0, 0, 0))]
    out_specs = []
    out_shapes = []
    if with_gru:
        out_specs.append(pl.BlockSpec((BN, D), lambda i: (i, 0)))
        out_shapes.append(jax.ShapeDtypeStruct((NPAD, D), jnp.float32))
    if with_ht:
        out_specs.append(pl.BlockSpec((T, BN, D), lambda i: (0, i, 0)))
        out_shapes.append(jax.ShapeDtypeStruct((T, NPAD, D), jnp.float32))
    if len(out_shapes) == 1:
        out_specs, out_shapes = out_specs[0], out_shapes[0]
    return pl.pallas_call(
        functools.partial(_tc_body, with_gru, with_ht),
        grid=grid,
        in_specs=in_specs,
        out_specs=out_specs,
        out_shape=out_shapes,
    )


_tc_ht0 = _make_tc_step(False, True)
_tc_gru_ht = _make_tc_step(True, True)
_tc_gru = _make_tc_step(True, False)

BNR = 640


def _readout_body(gid_ref, h_ref, out_ref):
    i = pl.program_id(0)
    ids = gid_ref[0]
    iot = lax.broadcasted_iota(jnp.int32, (G, BNR), 0)
    oh = (iot == ids).astype(jnp.float32)
    contrib = lax.dot_general(oh, h_ref[...], (((1,), (0,)), ((), ())),
                              precision=_HIGH,
                              preferred_element_type=jnp.float32)

    @pl.when(i == 0)
    def _():
        out_ref[...] = contrib

    @pl.when(i != 0)
    def _():
        out_ref[...] += contrib


_tc_readout = pl.pallas_call(
    _readout_body,
    grid=(NPAD // BNR,),
    in_specs=[
        pl.BlockSpec((1, 1, BNR), lambda i: (i, 0, 0)),
        pl.BlockSpec((BNR, D), lambda i: (i, 0)),
    ],
    out_specs=pl.BlockSpec((G, D), lambda i: (0, 0)),
    out_shape=jax.ShapeDtypeStruct((G, D), jnp.float32),
)


def kernel(features, edge_index, edge_types, graph_ids, W_edge, W_ih, W_hh, b_ih, b_hh):
    src = edge_index[0]
    dst = edge_index[1]
    pad_e = E_PAD - E
    gidx = edge_types * NPAD + src
    gidx = jnp.concatenate([gidx, jnp.full((pad_e,), N, jnp.int32)])
    didx = jnp.concatenate([dst, jnp.full((pad_e,), N, jnp.int32)])
    gidx3 = gidx.reshape(NCHTOT, CHUNK)
    didx3 = didx.reshape(NCHTOT, CHUNK)

    h = jnp.zeros((NPAD, D), jnp.float32).at[:N].set(features)
    zblk = jnp.zeros((CHUNK, D), jnp.float32)
    b_ih2 = b_ih.reshape(1, 3 * D)
    b_hh2 = b_hh.reshape(1, 3 * D)
    gid3 = (jnp.full((NPAD,), -1, jnp.int32).at[:N].set(graph_ids)
            .reshape(NPAD // BNR, 1, BNR))

    sc_scatter = _get_sc_scatter()
    ht = _tc_ht0(h, W_edge)
    for step in range(STEPS):
        m2 = sc_scatter(ht.reshape(T * NPAD, D), gidx3, didx3, zblk)
        if step < STEPS - 1:
            h, ht = _tc_gru_ht(m2, h, W_ih, W_hh, b_ih2, b_hh2, W_edge)
        else:
            h = _tc_gru(m2, h, W_ih, W_hh, b_ih2, b_hh2)
    return _tc_readout(gid3, h)
